# Initial kernel scaffold; baseline (speedup 1.0000x reference)
import jax
import jax.numpy as jnp
from jax.experimental import pallas as pl


def kernel(left, right, wl, bl, wr, br, wconv):
    raise NotImplementedError("write your pallas kernel here")


# speedup vs baseline: 6.7642x; 6.7642x over previous
import functools

import jax
import jax.numpy as jnp
from jax.experimental import pallas as pl
from jax.experimental.pallas import tpu as pltpu


def _cc_kernel(left_ref, right_ref, wl_ref, bl_ref, wr_ref, br_ref,
               wc0_ref, wc1_ref, wc2_ref, out_ref, *, L, NB, T):
    f32 = jnp.float32
    l2 = jnp.dot(wl_ref[...], left_ref[...],
                 preferred_element_type=f32) + bl_ref[...]
    r = jnp.dot(wr_ref[...], right_ref[...],
                preferred_element_type=f32) + br_ref[...]

    l23 = l2.reshape(NB, L, T)

    def l2row(j):
        return jnp.broadcast_to(l23[:, j:j + 1, :],
                                (NB, 2 * L, T)).reshape(NB * 2 * L, T)

    corr = l2row(0) * r
    for j in range(1, L):
        r = pltpu.roll(r, 1, axis=0)
        corr = corr + l2row(j) * r

    y0 = jnp.dot(wc0_ref[...], corr, preferred_element_type=f32)
    y1 = jnp.dot(wc1_ref[...], corr, preferred_element_type=f32)
    y2 = jnp.dot(wc2_ref[...], corr, preferred_element_type=f32)

    t = jax.lax.broadcasted_iota(jnp.int32, (1, T), 1)
    not_first = (t != 0).astype(f32)
    not_last = (t != T - 1).astype(f32)
    out_ref[...] = (y1
                    + not_first * pltpu.roll(y0, 1, axis=1)
                    + not_last * pltpu.roll(y2, T - 1, axis=1))


def _pick_nb(B, L, T):
    best = 1
    for nb in range(1, B + 1):
        if B % nb:
            continue
        rows = nb * L
        if rows > 128 or rows % 8:
            continue
        if nb * L * T * 4 > 2 * 1024 * 1024:
            continue
        best = nb
    return best


def kernel(left, right, wl, bl, wr, br, wconv):
    B, L, T = left.shape
    f32 = jnp.float32
    NB = _pick_nb(B, L, T)

    eye = jnp.eye(NB, dtype=f32)
    WL = jnp.kron(eye, wl.astype(f32))
    wr_g = jnp.concatenate([wr.astype(f32)[::-1, :],
                            jnp.zeros((L, L), f32)], axis=0)
    WR = jnp.kron(eye, wr_g)
    BL = jnp.tile(bl.astype(f32), NB).reshape(NB * L, 1)
    br_g = jnp.concatenate([br.astype(f32)[::-1],
                            jnp.zeros((L,), f32)], axis=0)
    BR = jnp.tile(br_g, NB).reshape(NB * 2 * L, 1)
    WC0 = jnp.kron(eye, wconv[:, :, 0].astype(f32))
    WC1 = jnp.kron(eye, wconv[:, :, 1].astype(f32))
    WC2 = jnp.kron(eye, wconv[:, :, 2].astype(f32))

    left2 = left.astype(f32).reshape(B * L, T)
    right2 = right.astype(f32).reshape(B * L, T)

    rows = NB * L
    io = pl.BlockSpec((rows, T), lambda i: (i, 0))
    cst = lambda shape: pl.BlockSpec(shape, lambda i: (0, 0))

    out2 = pl.pallas_call(
        functools.partial(_cc_kernel, L=L, NB=NB, T=T),
        out_shape=jax.ShapeDtypeStruct((B * L, T), f32),
        grid=(B // NB,),
        in_specs=[io, io,
                  cst((rows, rows)), cst((rows, 1)),
                  cst((2 * rows, rows)), cst((2 * rows, 1)),
                  cst((rows, 2 * rows)), cst((rows, 2 * rows)),
                  cst((rows, 2 * rows))],
        out_specs=io,
        compiler_params=pltpu.CompilerParams(
            dimension_semantics=("parallel",),
            vmem_limit_bytes=64 * 1024 * 1024),
    )(left2, right2, WL, BL, WR, BR, WC0, WC1, WC2)
    return out2.reshape(B, L, T)
